# baseline (speedup 1.0000x reference)
import functools
import jax
import jax.numpy as jnp
from jax import lax
from jax.experimental import pallas as pl
from jax.experimental.pallas import tpu as pltpu
from jax.experimental.pallas import tpu_sc as plsc

IN_C = 768
OUT_C = 384
_NC = 2
_NS = 16
_NW = _NC * _NS
_L = 16

_N = 16 * 56 * 56
_RPW = _N // _NW
_CH = 56
_NCHUNK = _RPW // _CH
_NBUF = 2


def _sc_body(x_hbm, mask_hbm, out_hbm, ibuf, obuf, maskv,
             isem0, isem1, osem0, osem1):
    isems = (isem0, isem1)
    osems = (osem0, osem1)
    wid = lax.axis_index("s") * _NC + lax.axis_index("c")
    base = wid * _RPW
    pltpu.sync_copy(mask_hbm.at[pl.ds(0, OUT_C)], maskv)
    mvecs = [maskv[pl.ds(j * _L, _L)] for j in range(OUT_C // _L)]

    def in_copy(chunk, b):
        return pltpu.make_async_copy(
            x_hbm.at[pl.ds(base + chunk * _CH, _CH), pl.ds(0, OUT_C)],
            ibuf.at[b], isems[b])

    def out_copy(chunk, b):
        return pltpu.make_async_copy(
            obuf.at[b],
            out_hbm.at[pl.ds(base + chunk * _CH, _CH)], osems[b])

    for b in range(_NBUF):
        in_copy(b, b).start()

    def group(g, carry):
        for b in range(_NBUF):
            chunk = g * _NBUF + b
            in_copy(chunk, b).wait()

            @pl.when(g > 0)
            def _wait_prev_out():
                out_copy(chunk - _NBUF, b).wait()

            def row(r, rcarry):
                for j in range(OUT_C // _L):
                    sl = pl.ds(j * _L, _L)
                    obuf[b, r, sl] = ibuf[b, r, sl] * mvecs[j]
                return rcarry

            lax.fori_loop(0, _CH, row, 0)
            out_copy(chunk, b).start()

            @pl.when(chunk + _NBUF < _NCHUNK)
            def _prefetch():
                in_copy(chunk + _NBUF, b).start()
        return carry

    lax.fori_loop(0, _NCHUNK // _NBUF, group, 0)
    for b in range(_NBUF):
        out_copy(_NCHUNK - _NBUF + b, b).wait()


@functools.cache
def _sc_call():
    mesh = plsc.VectorSubcoreMesh(
        core_axis_name="c", subcore_axis_name="s",
        num_cores=_NC, num_subcores=_NS)
    return pl.kernel(
        _sc_body,
        out_type=jax.ShapeDtypeStruct((_N, OUT_C), jnp.float32),
        mesh=mesh,
        scratch_types=[
            pltpu.VMEM((_NBUF, _CH, OUT_C), jnp.float32),
            pltpu.VMEM((_NBUF, _CH, OUT_C), jnp.float32),
            pltpu.VMEM((OUT_C,), jnp.float32),
            pltpu.SemaphoreType.DMA,
            pltpu.SemaphoreType.DMA,
            pltpu.SemaphoreType.DMA,
            pltpu.SemaphoreType.DMA,
        ],
    )


def kernel(x, mask):
    B, H, W, C = x.shape
    xf = x.reshape(B * H * W, C)
    mf = mask.reshape(C)
    out = _sc_call()(xf, mf)
    return out.reshape(B, H, W, OUT_C)
